# baseline (speedup 1.0000x reference)
import functools

import jax
import jax.numpy as jnp
from jax.experimental import pallas as pl
from jax.experimental.pallas import tpu as pltpu


def _make_body(B, T, H, D0, Op):
    TB = T * B

    def body(x_ref, wih_ref, whh_ref, bias_ref, fcw_ref, fcb_ref,
             out_ref, hcn_ref, g0_sc, h1_sc):

        g0_sc[...] = (
            jnp.dot(x_ref[...], wih_ref[0, :D0, :],
                    preferred_element_type=jnp.float32)
            + bias_ref[0])

        whh0 = whh_ref[0]
        wih1 = wih_ref[1, :H, :]
        whh1 = whh_ref[1]
        b1 = bias_ref[1]

        def cell(g, c):
            ifo = jax.nn.sigmoid(g[:, :3 * H])
            g_t = jnp.tanh(g[:, 3 * H:])
            i_g = ifo[:, :H]
            f_g = ifo[:, H:2 * H]
            o_g = ifo[:, 2 * H:]
            c_new = f_g * c + i_g * g_t
            h_new = o_g * jnp.tanh(c_new)
            return h_new, c_new

        def l0_step(t, h0, c0):
            r0 = pl.multiple_of(t * B, B)
            g = g0_sc[pl.ds(r0, B), :] + jnp.dot(
                h0.astype(jnp.bfloat16), whh0, preferred_element_type=jnp.float32)
            return cell(g, c0)

        def l1_step(t, x1, h1, c1):
            g = (jnp.dot(x1, wih1, preferred_element_type=jnp.float32)
                 + jnp.dot(h1.astype(jnp.bfloat16), whh1,
                           preferred_element_type=jnp.float32)
                 + b1)
            h_new, c_new = cell(g, c1)
            r0 = pl.multiple_of(t * B, B)
            h1_sc[pl.ds(r0, B), :] = h_new.astype(jnp.bfloat16)
            return h_new, c_new

        h0, c0 = cell(g0_sc[pl.ds(0, B), :], jnp.zeros((B, H), jnp.float32))

        def l1_first(x1):
            g = jnp.dot(x1, wih1, preferred_element_type=jnp.float32) + b1
            h_new, c_new = cell(g, jnp.zeros((B, H), jnp.float32))
            h1_sc[pl.ds(0, B), :] = h_new.astype(jnp.bfloat16)
            return h_new, c_new

        h1 = c1 = None
        for i in range(1, T):
            h0_prev = h0.astype(jnp.bfloat16)
            h0, c0 = l0_step(i, h0, c0)
            if i == 1:
                h1, c1 = l1_first(h0_prev)
            else:
                h1, c1 = l1_step(i - 1, h0_prev, h1, c1)
        h1, c1 = l1_step(T - 1, h0.astype(jnp.bfloat16), h1, c1)

        hcn_ref[0] = jnp.concatenate([h0, c0], axis=-1)
        hcn_ref[1] = jnp.concatenate([h1, c1], axis=-1)

        out_ref[...] = (
            jnp.dot(h1_sc[...], fcw_ref[...], preferred_element_type=jnp.float32)
            + fcb_ref[...])

    return body


@functools.partial(jax.jit, static_argnames=("B", "T", "H", "D0", "Op"))
def _forward(x_tm, wih, whh, bias, fcw, fcb, *, B, T, H, D0, Op):
    TB = T * B
    Din = wih.shape[1]
    body = _make_body(B, T, H, D0, Op)

    vmem_bytes = (
        TB * D0 * 2
        + TB * Op * 4
        + TB * 4 * H * 4
        + TB * H * 2
        + 2 * Din * 4 * H * 2
        + 2 * H * 4 * H * 2
        + 2 * 4 * H * 4
        + H * Op * 2 + Op * 4
        + 2 * B * 2 * H * 4)
    vmem_limit = int(min(vmem_bytes + (16 << 20), 60000 * 1024))

    return pl.pallas_call(
        body,
        out_shape=(
            jax.ShapeDtypeStruct((TB, Op), jnp.float32),
            jax.ShapeDtypeStruct((2, B, 2 * H), jnp.float32),
        ),
        scratch_shapes=[
            pltpu.VMEM((TB, 4 * H), jnp.float32),
            pltpu.VMEM((TB, H), jnp.bfloat16),
        ],
        compiler_params=pltpu.CompilerParams(
            vmem_limit_bytes=vmem_limit),
    )(x_tm, wih, whh, bias, fcw, fcb)


def kernel(x, wih, whh, bias, fc_w_t, fc_b):
    B, T, D0 = x.shape
    L, Din, fourH = wih.shape
    H = fourH // 4
    Op = fc_w_t.shape[1]
    O = fc_b.shape[1]

    x_tm = jnp.swapaxes(x, 0, 1).reshape(T * B, D0).astype(jnp.bfloat16)
    fcb_p = fc_b if Op == O else jnp.pad(fc_b, ((0, 0), (0, Op - O)))

    out2d, hcn = _forward(
        x_tm, wih, whh, bias, fc_w_t, fcb_p,
        B=B, T=T, H=H, D0=D0, Op=Op)

    out = jnp.swapaxes(out2d.reshape(T, B, Op), 0, 1)[:, :, :O]
    hn = hcn[:, :, :H]
    cn = hcn[:, :, H:]
    return out, (hn, cn)


# speedup vs baseline: 1.8044x; 1.8044x over previous
import functools

import jax
import jax.numpy as jnp
from jax.experimental import pallas as pl
from jax.experimental.pallas import tpu as pltpu


def _make_body(B, T, H, D0, Op, NT):
    TB = T * B
    Tc = T // NT

    def body(x_ref, wih_ref, whh_ref, bias_ref, fcw_ref, fcb_ref,
             out_ref, hn_ref, cn_ref, g0_sc, h1_sc):

        wih0 = wih_ref[0, :D0, :]

        for tc in range(NT):
            chunk = x_ref[:, tc * Tc:(tc + 1) * Tc, :]
            sw = jnp.swapaxes(chunk, 0, 1).reshape(Tc * B, D0)
            g0_sc[pl.ds(tc * Tc * B, Tc * B), :] = (
                jnp.dot(sw.astype(jnp.bfloat16), wih0,
                        preferred_element_type=jnp.float32)
                + bias_ref[0])

        whh0 = whh_ref[0]
        wih1 = wih_ref[1, :H, :]
        whh1 = whh_ref[1]
        b1 = bias_ref[1]

        def cell(g, c):
            ifo = jax.nn.sigmoid(g[:, :3 * H])
            g_t = jnp.tanh(g[:, 3 * H:])
            i_g = ifo[:, :H]
            f_g = ifo[:, H:2 * H]
            o_g = ifo[:, 2 * H:]
            c_new = f_g * c + i_g * g_t
            h_new = o_g * jnp.tanh(c_new)
            return h_new, c_new

        def l0_step(t, h0, c0):
            r0 = pl.multiple_of(t * B, B)
            g = g0_sc[pl.ds(r0, B), :] + jnp.dot(
                h0.astype(jnp.bfloat16), whh0, preferred_element_type=jnp.float32)
            return cell(g, c0)

        def l1_step(t, x1, h1, c1):
            g = (jnp.dot(x1, wih1, preferred_element_type=jnp.float32)
                 + jnp.dot(h1.astype(jnp.bfloat16), whh1,
                           preferred_element_type=jnp.float32)
                 + b1)
            h_new, c_new = cell(g, c1)
            r0 = pl.multiple_of(t * B, B)
            h1_sc[pl.ds(r0, B), :] = h_new.astype(jnp.bfloat16)
            return h_new, c_new

        h0, c0 = cell(g0_sc[pl.ds(0, B), :], jnp.zeros((B, H), jnp.float32))

        def l1_first(x1):
            g = jnp.dot(x1, wih1, preferred_element_type=jnp.float32) + b1
            h_new, c_new = cell(g, jnp.zeros((B, H), jnp.float32))
            h1_sc[pl.ds(0, B), :] = h_new.astype(jnp.bfloat16)
            return h_new, c_new

        h1 = c1 = None
        for i in range(1, T):
            h0_prev = h0.astype(jnp.bfloat16)
            h0, c0 = l0_step(i, h0, c0)
            if i == 1:
                h1, c1 = l1_first(h0_prev)
            else:
                h1, c1 = l1_step(i - 1, h0_prev, h1, c1)
        h1, c1 = l1_step(T - 1, h0.astype(jnp.bfloat16), h1, c1)

        hn_ref[0] = h0
        hn_ref[1] = h1
        cn_ref[0] = c0
        cn_ref[1] = c1

        fcw = fcw_ref[...]
        fcb = fcb_ref[...]
        for tc in range(NT):
            fc = (jnp.dot(h1_sc[pl.ds(tc * Tc * B, Tc * B), :], fcw,
                          preferred_element_type=jnp.float32)
                  + fcb)
            out_ref[:, tc * Tc:(tc + 1) * Tc, :] = (
                jnp.swapaxes(fc.reshape(Tc, B, Op), 0, 1))

    return body


@functools.partial(jax.jit, static_argnames=("B", "T", "H", "D0", "Op", "NT"))
def _forward(x, wih, whh, bias, fcw, fcb, *, B, T, H, D0, Op, NT):
    TB = T * B
    Din = wih.shape[1]
    body = _make_body(B, T, H, D0, Op, NT)

    vmem_bytes = (
        TB * D0 * 4
        + TB * Op * 4
        + TB * 4 * H * 4
        + TB * H * 2
        + 2 * Din * 4 * H * 2
        + 2 * H * 4 * H * 2
        + 2 * 4 * H * 4
        + H * Op * 2 + Op * 4
        + 4 * B * H * 4)
    vmem_limit = int(min(vmem_bytes + (12 << 20), 60000 * 1024))

    return pl.pallas_call(
        body,
        out_shape=(
            jax.ShapeDtypeStruct((B, T, Op), jnp.float32),
            jax.ShapeDtypeStruct((2, B, H), jnp.float32),
            jax.ShapeDtypeStruct((2, B, H), jnp.float32),
        ),
        scratch_shapes=[
            pltpu.VMEM((TB, 4 * H), jnp.float32),
            pltpu.VMEM((TB, H), jnp.bfloat16),
        ],
        compiler_params=pltpu.CompilerParams(
            vmem_limit_bytes=vmem_limit),
    )(x, wih, whh, bias, fcw, fcb)


def kernel(x, wih, whh, bias, fc_w_t, fc_b):
    B, T, D0 = x.shape
    L, Din, fourH = wih.shape
    H = fourH // 4
    Op = fc_w_t.shape[1]
    O = fc_b.shape[1]
    NT = 8 if T % 8 == 0 else 1

    fcb_p = fc_b if Op == O else jnp.pad(fc_b, ((0, 0), (0, Op - O)))

    out, hn, cn = _forward(
        x, wih, whh, bias, fc_w_t, fcb_p,
        B=B, T=T, H=H, D0=D0, Op=Op, NT=NT)

    if Op != O:
        out = out[:, :, :O]
    return out, (hn, cn)
